# Initial kernel scaffold; baseline (speedup 1.0000x reference)
import jax
import jax.numpy as jnp
from jax.experimental import pallas as pl


def kernel(x, edge_attr, W_edge_init, b_edge_init, W_conv0, b_conv0, W_conv1, b_conv1, W_conv2, b_conv2, W_e2n, b_e2n, W_ffn, b_ffn, edge_index, batch):
    raise NotImplementedError("write your pallas kernel here")


# speedup vs baseline: 3.1892x; 3.1892x over previous
import functools

import jax
import jax.numpy as jnp
from jax import lax
from jax.experimental import pallas as pl
from jax.experimental.pallas import tpu as pltpu
from jax.experimental.pallas import tpu_sc as plsc

N = 10000
E = 640000
H = 128
DN = 128
DE = 16
G = 64

NC = 2
NS = 16
NW = NC * NS

GRP = 128
SUP = 1024
NSUP = E // SUP
CH = 256
NCHK = SUP // CH
RSUB = 632
RLAST = N - 15 * RSUB

_mesh = plsc.VectorSubcoreMesh(core_axis_name="c", subcore_axis_name="s")


@functools.partial(
    pl.kernel,
    out_type=jax.ShapeDtypeStruct((E, H), jnp.float32),
    mesh=_mesh,
    scratch_types=[
        pltpu.VMEM((SUP // GRP, GRP), jnp.int32),
        pltpu.VMEM((CH, H), jnp.float32),
    ],
)
def _sc_gather(table_hbm, idx_hbm, out_hbm, idx_v, buf_v):
    w = lax.axis_index("s") * NC + lax.axis_index("c")
    niter = (NSUP - w + NW - 1) // NW

    def body(k, _):
        j = w + k * NW
        pltpu.sync_copy(idx_hbm.at[j], idx_v)
        for t in range(NCHK):
            for u in range(CH // GRP):
                pltpu.sync_copy(table_hbm.at[idx_v.at[t * (CH // GRP) + u]],
                                buf_v.at[pl.ds(u * GRP, GRP)])
            pltpu.sync_copy(buf_v, out_hbm.at[pl.ds(j * SUP + t * CH, CH)])
        return 0

    lax.fori_loop(0, niter, body, 0)


@functools.partial(
    pl.kernel,
    out_type=jax.ShapeDtypeStruct((NC, N, H), jnp.float32),
    mesh=_mesh,
    scratch_types=[
        pltpu.VMEM((SUP // GRP, GRP), jnp.int32),
        pltpu.VMEM((CH, H), jnp.float32),
        pltpu.VMEM_SHARED((N, H), jnp.float32),
    ],
)
def _sc_scatter(g_hbm, idx_hbm, zero_hbm, out_hbm, idx_v, buf_v, table_sh):
    c = lax.axis_index("c")
    s = lax.axis_index("s")
    wid = s * NC + c

    @pl.when(s < NS - 1)
    def _():
        pltpu.sync_copy(zero_hbm.at[pl.ds(s * RSUB, RSUB)],
                        table_sh.at[pl.ds(s * RSUB, RSUB)])

    @pl.when(s == NS - 1)
    def _():
        pltpu.sync_copy(zero_hbm.at[pl.ds(15 * RSUB, RLAST)],
                        table_sh.at[pl.ds(15 * RSUB, RLAST)])

    plsc.subcore_barrier()

    niter = (NSUP - wid + NW - 1) // NW

    def body(k, _):
        j = wid + k * NW
        pltpu.sync_copy(idx_hbm.at[j], idx_v)
        for t in range(NCHK):
            pltpu.sync_copy(g_hbm.at[pl.ds(j * SUP + t * CH, CH)], buf_v)
            for u in range(CH // GRP):
                pltpu.sync_copy(buf_v.at[pl.ds(u * GRP, GRP)],
                                table_sh.at[idx_v.at[t * (CH // GRP) + u]],
                                add=True)
        return 0

    lax.fori_loop(0, niter, body, 0)
    plsc.subcore_barrier()

    @pl.when(s < NS - 1)
    def _():
        pltpu.sync_copy(table_sh.at[pl.ds(s * RSUB, RSUB)],
                        out_hbm.at[c, pl.ds(s * RSUB, RSUB)])

    @pl.when(s == NS - 1)
    def _():
        pltpu.sync_copy(table_sh.at[pl.ds(15 * RSUB, RLAST)],
                        out_hbm.at[c, pl.ds(15 * RSUB, RLAST)])


def _dot(a, b):
    return jnp.dot(a, b, preferred_element_type=jnp.float32)


def _tc_xw_body(x_ref, w_ref, o_ref):
    o_ref[...] = _dot(x_ref[...], w_ref[...])


def _tc_xw(x, w):
    return pl.pallas_call(
        _tc_xw_body,
        out_shape=jax.ShapeDtypeStruct((N, H), jnp.float32),
    )(x, w)


def _tc_add_body(p_ref, o_ref):
    o_ref[...] = p_ref[0] + p_ref[1]


def _tc_add(p):
    return pl.pallas_call(
        _tc_add_body,
        out_shape=jax.ShapeDtypeStruct((N, H), jnp.float32),
    )(p)


_B0 = 2000


def _tc_edge_init_body(xr_ref, ea_ref, we_ref, bei_ref, w0_ref,
                       h0_ref, g0_ref):
    h0 = jax.nn.relu(xr_ref[...] + _dot(ea_ref[...], we_ref[...])
                     + bei_ref[...])
    h0_ref[...] = h0
    g0_ref[...] = _dot(h0, w0_ref[...])


def _tc_edge_init(xr, ea, we, bei, w0):
    return pl.pallas_call(
        _tc_edge_init_body,
        grid=(E // _B0,),
        in_specs=[
            pl.BlockSpec((_B0, H), lambda i: (i, 0)),
            pl.BlockSpec((_B0, DE), lambda i: (i, 0)),
            pl.BlockSpec((DE, H), lambda i: (0, 0)),
            pl.BlockSpec((1, H), lambda i: (0, 0)),
            pl.BlockSpec((H, H), lambda i: (0, 0)),
        ],
        out_specs=[
            pl.BlockSpec((_B0, H), lambda i: (i, 0)),
            pl.BlockSpec((_B0, H), lambda i: (i, 0)),
        ],
        out_shape=[
            jax.ShapeDtypeStruct((E, H), jnp.float32),
            jax.ShapeDtypeStruct((E, H), jnp.float32),
        ],
    )(xr, ea, we, bei, w0)


_BP = 1000


def _tc_conv_body(ar_ref, g_ref, h0_ref, w_ref, b_ref, o_ref):
    ge = g_ref[:, 0, :]
    go = g_ref[:, 1, :]
    he = jax.nn.relu(ar_ref[:, 0, :] - go + b_ref[...] + h0_ref[:, 0, :])
    ho = jax.nn.relu(ar_ref[:, 1, :] - ge + b_ref[...] + h0_ref[:, 1, :])
    o_ref[:, 0, :] = _dot(he, w_ref[...])
    o_ref[:, 1, :] = _dot(ho, w_ref[...])


def _tc_conv_last_body(ar_ref, g_ref, h0_ref, b_ref, o_ref):
    ge = g_ref[:, 0, :]
    go = g_ref[:, 1, :]
    o_ref[:, 0, :] = jax.nn.relu(ar_ref[:, 0, :] - go + b_ref[...]
                                 + h0_ref[:, 0, :])
    o_ref[:, 1, :] = jax.nn.relu(ar_ref[:, 1, :] - ge + b_ref[...]
                                 + h0_ref[:, 1, :])


def _pair_spec():
    return pl.BlockSpec((_BP, 2, H), lambda i: (i, 0, 0))


def _tc_conv(ar3, g3, h03, w, b):
    return pl.pallas_call(
        _tc_conv_body,
        grid=(E // 2 // _BP,),
        in_specs=[
            _pair_spec(), _pair_spec(), _pair_spec(),
            pl.BlockSpec((H, H), lambda i: (0, 0)),
            pl.BlockSpec((1, H), lambda i: (0, 0)),
        ],
        out_specs=_pair_spec(),
        out_shape=jax.ShapeDtypeStruct((E // 2, 2, H), jnp.float32),
    )(ar3, g3, h03, w, b)


def _tc_conv_last(ar3, g3, h03, b):
    return pl.pallas_call(
        _tc_conv_last_body,
        grid=(E // 2 // _BP,),
        in_specs=[
            _pair_spec(), _pair_spec(), _pair_spec(),
            pl.BlockSpec((1, H), lambda i: (0, 0)),
        ],
        out_specs=_pair_spec(),
        out_shape=jax.ShapeDtypeStruct((E // 2, 2, H), jnp.float32),
    )(ar3, g3, h03, b)


_BN = 1000


def _tc_final_body(x_ref, s_ref, b2_ref, w1_ref, w2_ref, be_ref, wf_ref,
                   bf_ref, o_ref, acc_ref):
    i = pl.program_id(0)

    @pl.when(i == 0)
    def _():
        acc_ref[...] = jnp.zeros_like(acc_ref)

    hn = jax.nn.relu(_dot(x_ref[...], w1_ref[...])
                     + _dot(s_ref[...], w2_ref[...]) + be_ref[...])
    onehot = (b2_ref[...] == lax.broadcasted_iota(jnp.int32, (_BN, G), 1))
    acc_ref[...] += lax.dot_general(
        onehot.astype(jnp.float32), hn,
        (((0,), (0,)), ((), ())), preferred_element_type=jnp.float32)

    @pl.when(i == pl.num_programs(0) - 1)
    def _():
        o_ref[...] = (jnp.sum(acc_ref[...] * wf_ref[...], axis=1,
                              keepdims=True) + bf_ref[...])


def _tc_final(x, s, batch2, w1, w2, be, wf_row, bf):
    return pl.pallas_call(
        _tc_final_body,
        grid=(N // _BN,),
        in_specs=[
            pl.BlockSpec((_BN, DN), lambda i: (i, 0)),
            pl.BlockSpec((_BN, H), lambda i: (i, 0)),
            pl.BlockSpec((_BN, 1), lambda i: (i, 0)),
            pl.BlockSpec((DN, H), lambda i: (0, 0)),
            pl.BlockSpec((H, H), lambda i: (0, 0)),
            pl.BlockSpec((1, H), lambda i: (0, 0)),
            pl.BlockSpec((1, H), lambda i: (0, 0)),
            pl.BlockSpec((1, 1), lambda i: (0, 0)),
        ],
        out_specs=pl.BlockSpec((G, 1), lambda i: (0, 0)),
        out_shape=jax.ShapeDtypeStruct((G, 1), jnp.float32),
        scratch_shapes=[pltpu.VMEM((G, H), jnp.float32)],
    )(x, s, batch2, w1, w2, be, wf_row, bf)


def kernel(x, edge_attr, W_edge_init, b_edge_init, W_conv0, b_conv0,
           W_conv1, b_conv1, W_conv2, b_conv2, W_e2n, b_e2n, W_ffn, b_ffn,
           edge_index, batch):
    row = edge_index[0].astype(jnp.int32)
    col = edge_index[1].astype(jnp.int32)
    row2 = row.reshape(NSUP, SUP // GRP, GRP)
    col2 = col.reshape(NSUP, SUP // GRP, GRP)
    zeros_n = jnp.zeros((N, H), jnp.float32)

    bei = b_edge_init.reshape(1, H)
    b0 = b_conv0.reshape(1, H)
    b1 = b_conv1.reshape(1, H)
    b2 = b_conv2.reshape(1, H)
    be = b_e2n.reshape(1, H)
    wf_row = W_ffn.reshape(1, H)
    bf = b_ffn.reshape(1, 1)
    batch2 = batch.astype(jnp.int32).reshape(N, 1)

    xw = _tc_xw(x, W_edge_init[:DN])
    xr = _sc_gather(xw, row2)
    h0, g0 = _tc_edge_init(xr, edge_attr, W_edge_init[DN:], bei, W_conv0)

    h03 = h0.reshape(E // 2, 2, H)
    g = g0
    for w_next, b_cur in ((W_conv1, b0), (W_conv2, b1)):
        a = _tc_add(_sc_scatter(g, col2, zeros_n))
        ar3 = _sc_gather(a, row2).reshape(E // 2, 2, H)
        g = _tc_conv(ar3, g.reshape(E // 2, 2, H), h03, w_next, b_cur)
        g = g.reshape(E, H)

    a = _tc_add(_sc_scatter(g, col2, zeros_n))
    ar3 = _sc_gather(a, row2).reshape(E // 2, 2, H)
    h3 = _tc_conv_last(ar3, g.reshape(E // 2, 2, H), h03, b2)

    s = _tc_add(_sc_scatter(h3.reshape(E, H), col2, zeros_n))
    out = _tc_final(x, s, batch2, W_e2n[:DN], W_e2n[DN:], be, wf_row, bf)
    return out.reshape(G)
